# baseline (device time: 9708 ns/iter reference)
import jax
import jax.numpy as jnp
from jax import lax
from jax.experimental import pallas as pl
from jax.experimental.pallas import tpu as pltpu

K = 4


def kernel(x):
    _, m, n2 = x.shape
    n = n2 // 2
    half = m // 2
    c = half // K

    def body(x_ref, out_ref, comm_ref, send_x, recv_x, send_y, recv_y):
        my_x = lax.axis_index("x")
        my_y = lax.axis_index("y")
        other_x = 1 - my_x
        other_y = 1 - my_y
        row0 = my_y * half
        row1 = other_y * half

        barrier_sem = pltpu.get_barrier_semaphore()
        for dev in ((other_x, my_y), (my_x, other_y)):
            pl.semaphore_signal(
                barrier_sem, inc=1,
                device_id=dev, device_id_type=pl.DeviceIdType.MESH,
            )
        pl.semaphore_wait(barrier_sem, 2)

        def x_rdma(j):
            return pltpu.make_async_remote_copy(
                src_ref=x_ref.at[0, pl.ds(row0 + j * c, c), pl.ds(other_x * n, n)],
                dst_ref=comm_ref.at[pl.ds(row0 + j * c, c), :],
                send_sem=send_x.at[j],
                recv_sem=recv_x.at[j],
                device_id=(other_x, my_y),
                device_id_type=pl.DeviceIdType.MESH,
            )

        def y_send(j):
            return pltpu.make_async_remote_copy(
                src_ref=comm_ref.at[pl.ds(row0 + j * c, c), :],
                dst_ref=comm_ref.at[pl.ds(row0 + j * c, c), :],
                send_sem=send_y.at[j],
                recv_sem=recv_y.at[j],
                device_id=(my_x, other_y),
                device_id_type=pl.DeviceIdType.MESH,
            )

        def y_recv(j):
            return pltpu.make_async_remote_copy(
                src_ref=comm_ref.at[pl.ds(row1 + j * c, c), :],
                dst_ref=comm_ref.at[pl.ds(row1 + j * c, c), :],
                send_sem=send_y.at[j],
                recv_sem=recv_y.at[j],
                device_id=(my_x, other_y),
                device_id_type=pl.DeviceIdType.MESH,
            )

        for j in range(K):
            x_rdma(j).start()

        out_ref[:, :] = x_ref[0, :, pl.ds(my_x * n, n)]

        for j in range(K):
            x_rdma(j).wait_recv()
            y_send(j).start()

        out_ref[pl.ds(row0, half), :] = (
            out_ref[pl.ds(row0, half), :] + comm_ref[pl.ds(row0, half), :]
        )

        for j in range(K):
            y_recv(j).wait_recv()
        out_ref[pl.ds(row1, half), :] = (
            out_ref[pl.ds(row1, half), :] + comm_ref[pl.ds(row1, half), :]
        )

        for j in range(K):
            x_rdma(j).wait_send()
            y_send(j).wait_send()

    return pl.pallas_call(
        body,
        out_shape=jax.ShapeDtypeStruct((m, n), jnp.float32),
        in_specs=[pl.BlockSpec(memory_space=pltpu.VMEM)],
        out_specs=pl.BlockSpec(memory_space=pltpu.VMEM),
        scratch_shapes=[
            pltpu.VMEM((m, n), jnp.float32),
            pltpu.SemaphoreType.DMA((K,)),
            pltpu.SemaphoreType.DMA((K,)),
            pltpu.SemaphoreType.DMA((K,)),
            pltpu.SemaphoreType.DMA((K,)),
        ],
        compiler_params=pltpu.CompilerParams(collective_id=0),
    )(x)


# device time: 8304 ns/iter; 1.1691x vs baseline; 1.1691x over previous
import jax
import jax.numpy as jnp
from jax import lax
from jax.experimental import pallas as pl
from jax.experimental.pallas import tpu as pltpu


def kernel(x):
    _, m, n2 = x.shape
    n = n2 // 2

    def body(x_hbm, out_hbm, x_send, x_loc, out_v, comm,
             ld_send_sem, ld_loc_sem, st_sem, send_sem, recv_sem):
        my_x = lax.axis_index("x")
        my_y = lax.axis_index("y")
        other_x = 1 - my_x

        ld_send = pltpu.make_async_copy(
            x_hbm.at[0, :, pl.ds(other_x * n, n)], x_send, ld_send_sem)
        ld_send.start()
        ld_loc = pltpu.make_async_copy(
            x_hbm.at[0, :, pl.ds(my_x * n, n)], x_loc, ld_loc_sem)
        ld_loc.start()

        barrier_sem = pltpu.get_barrier_semaphore()
        pl.semaphore_signal(
            barrier_sem, inc=1,
            device_id=(other_x, my_y), device_id_type=pl.DeviceIdType.MESH,
        )
        pl.semaphore_wait(barrier_sem, 1)

        ld_send.wait()
        rdma = pltpu.make_async_remote_copy(
            src_ref=x_send,
            dst_ref=comm,
            send_sem=send_sem,
            recv_sem=recv_sem,
            device_id=(other_x, my_y),
            device_id_type=pl.DeviceIdType.MESH,
        )
        rdma.start()

        ld_loc.wait()
        rdma.wait_recv()
        out_v[:, :] = x_loc[:, :] + comm[:, :]

        st = pltpu.make_async_copy(out_v, out_hbm, st_sem)
        st.start()
        st.wait()
        rdma.wait_send()

    return pl.pallas_call(
        body,
        out_shape=jax.ShapeDtypeStruct((m, n), jnp.float32),
        in_specs=[pl.BlockSpec(memory_space=pl.ANY)],
        out_specs=pl.BlockSpec(memory_space=pl.ANY),
        scratch_shapes=[
            pltpu.VMEM((m, n), jnp.float32),
            pltpu.VMEM((m, n), jnp.float32),
            pltpu.VMEM((m, n), jnp.float32),
            pltpu.VMEM((m, n), jnp.float32),
            pltpu.SemaphoreType.DMA,
            pltpu.SemaphoreType.DMA,
            pltpu.SemaphoreType.DMA,
            pltpu.SemaphoreType.DMA,
            pltpu.SemaphoreType.DMA,
        ],
        compiler_params=pltpu.CompilerParams(collective_id=0),
    )(x)


# device time: 8217 ns/iter; 1.1815x vs baseline; 1.0106x over previous
import jax
import jax.numpy as jnp
from jax import lax
from jax.experimental import pallas as pl
from jax.experimental.pallas import tpu as pltpu


def kernel(x):
    _, m, n2 = x.shape
    n = n2 // 2

    def body(x_ref, out_ref, comm_ref, send_sem, recv_sem):
        my_x = lax.axis_index("x")
        my_y = lax.axis_index("y")
        other_x = 1 - my_x

        barrier_sem = pltpu.get_barrier_semaphore()
        pl.semaphore_signal(
            barrier_sem, inc=1,
            device_id=(other_x, my_y), device_id_type=pl.DeviceIdType.MESH,
        )
        pl.semaphore_wait(barrier_sem, 1)

        rdma = pltpu.make_async_remote_copy(
            src_ref=x_ref.at[0, :, pl.ds(other_x * n, n)],
            dst_ref=comm_ref,
            send_sem=send_sem,
            recv_sem=recv_sem,
            device_id=(other_x, my_y),
            device_id_type=pl.DeviceIdType.MESH,
        )
        rdma.start()

        @pl.when(my_x == 0)
        def _():
            out_ref[:, :] = x_ref[0, :, :n]

        @pl.when(my_x == 1)
        def _():
            out_ref[:, :] = x_ref[0, :, n:]

        rdma.wait_recv()
        out_ref[:, :] = out_ref[:, :] + comm_ref[:, :]
        rdma.wait_send()

    return pl.pallas_call(
        body,
        out_shape=jax.ShapeDtypeStruct((m, n), jnp.float32),
        in_specs=[pl.BlockSpec(memory_space=pltpu.VMEM)],
        out_specs=pl.BlockSpec(memory_space=pltpu.VMEM),
        scratch_shapes=[
            pltpu.VMEM((m, n), jnp.float32),
            pltpu.SemaphoreType.DMA,
            pltpu.SemaphoreType.DMA,
        ],
        compiler_params=pltpu.CompilerParams(collective_id=0),
    )(x)
